# Initial kernel scaffold; baseline (speedup 1.0000x reference)
import jax
import jax.numpy as jnp
from jax.experimental import pallas as pl


def kernel(x, target, p_target, e_target, mel_max_length, params, pitch_bins, energy_bins):
    raise NotImplementedError("write your pallas kernel here")


# speedup vs baseline: 4.1376x; 4.1376x over previous
import functools

import jax
import jax.numpy as jnp
from jax import lax
from jax.experimental import pallas as pl
from jax.experimental.pallas import tpu as pltpu
from jax.experimental.pallas import tpu_sc as plsc

_ENC = 256
_NBINS = 256
_NB = 4
_LP = 512
_LM = 1536
_ZROW = _NB * _LP
_ROWS = _NB * _LM
_NW = 32
_RPW = _ROWS // _NW
_CH = 96
_NCH = _RPW // _CH


def _idx_body(mel_ref, tgt_ref, pt_ref, et_ref, pbins_ref, ebins_ref,
              src_ref, pidx_ref, eidx_ref):
    b = pl.program_id(0)
    dur = tgt_ref[0, 0, :].reshape(1, _LP)
    tri = (lax.broadcasted_iota(jnp.int32, (_LP, _LP), 0)
           <= lax.broadcasted_iota(jnp.int32, (_LP, _LP), 1)).astype(jnp.float32)
    c = jnp.dot(dur, tri, preferred_element_type=jnp.float32).astype(jnp.int32)
    tot = jnp.minimum(c[0, _LP - 1], mel_ref[0])
    j = lax.broadcasted_iota(jnp.int32, (_LM, 1), 0)
    cnt = jnp.sum((c <= j).astype(jnp.int32), axis=1)
    valid = j[:, 0] < tot
    src_ref[0, 0, :] = jnp.where(valid, b * _LP + cnt, _ZROW)
    pt = pt_ref[0, 0, :]
    pidx_ref[0, 0, :] = jnp.sum((pbins_ref[:][None, :] < pt[:, None]).astype(jnp.int32), axis=1)
    et = et_ref[0, 0, :]
    eidx_ref[0, 0, :] = jnp.sum((ebins_ref[:][None, :] < et[:, None]).astype(jnp.int32), axis=1)


@functools.lru_cache(maxsize=1)
def _sc_gather_fn():
    mesh = plsc.VectorSubcoreMesh(core_axis_name="c", subcore_axis_name="s")

    @functools.partial(
        pl.kernel,
        mesh=mesh,
        out_type=[
            jax.ShapeDtypeStruct((_ROWS, _ENC), jnp.float32),
            jax.ShapeDtypeStruct((_ROWS, _ENC), jnp.float32),
        ],
        scratch_types=[
            pltpu.VMEM((_CH,), jnp.int32),
            pltpu.VMEM((_CH,), jnp.int32),
            pltpu.VMEM((_CH,), jnp.int32),
            pltpu.VMEM((_CH, _ENC), jnp.float32),
            pltpu.VMEM((_CH, _ENC), jnp.float32),
            pltpu.VMEM((_CH, _ENC), jnp.float32),
            pltpu.SemaphoreType.DMA,
        ],
    )
    def _sc_gather(xz, ptab, etab, src, pidx, eidx, out0, outf,
                   src_v, pidx_v, eidx_v, xr, pr, er, sem):
        _sc_gather_body(xz, ptab, etab, src, pidx, eidx, out0, outf,
                        src_v, pidx_v, eidx_v, xr, pr, er, sem)

    return _sc_gather


def _sc_gather_body(xz, ptab, etab, src, pidx, eidx, out0, outf,
                    src_v, pidx_v, eidx_v, xr, pr, er, sem):
    wid = lax.axis_index("s") * 2 + lax.axis_index("c")
    base = wid * _RPW

    def chunk(k, carry):
        off = base + k * _CH
        pltpu.sync_copy(src.at[pl.ds(off, _CH)], src_v)
        pltpu.sync_copy(pidx.at[pl.ds(off, _CH)], pidx_v)
        pltpu.sync_copy(eidx.at[pl.ds(off, _CH)], eidx_v)
        cx = pltpu.async_copy(xz.at[src_v], xr, sem)
        cp = pltpu.async_copy(ptab.at[pidx_v], pr, sem)
        ce = pltpu.async_copy(etab.at[eidx_v], er, sem)
        cx.wait()
        cp.wait()
        ce.wait()
        pltpu.sync_copy(xr, out0.at[pl.ds(off, _CH)])

        def row(r, carry2):
            for q in range(_ENC // 16):
                sl = pl.ds(q * 16, 16)
                xr[r, sl] = xr[r, sl] + pr[r, sl] + er[r, sl]
            return carry2

        lax.fori_loop(0, _CH, row, 0)
        pltpu.sync_copy(xr, outf.at[pl.ds(off, _CH)])
        return carry

    lax.fori_loop(0, _NCH, chunk, 0)


def _conv_relu_ln(h, w_ref, b_ref, g_ref, bt_ref):
    a0 = jnp.dot(h, w_ref[0], preferred_element_type=jnp.float32)
    a1 = jnp.dot(h, w_ref[1], preferred_element_type=jnp.float32)
    a2 = jnp.dot(h, w_ref[2], preferred_element_type=jnp.float32)
    z = jnp.zeros((1, _ENC), jnp.float32)
    y = (a1 + jnp.concatenate([z, a0[:-1]], axis=0)
         + jnp.concatenate([a2[1:], z], axis=0) + b_ref[:][None, :])
    y = jnp.maximum(y, 0.0)
    m = jnp.mean(y, axis=1, keepdims=True)
    v = jnp.mean(jnp.square(y - m), axis=1, keepdims=True)
    return (y - m) * lax.rsqrt(v + 1e-5) * g_ref[:][None, :] + bt_ref[:][None, :]


def _vp(h, refs, lb):
    c1w, c1b, l1g, l1b, c2w, c2b, l2g, l2b, lwv = refs
    h1 = _conv_relu_ln(h, c1w, c1b, l1g, l1b)
    h2 = _conv_relu_ln(h1, c2w, c2b, l2g, l2b)
    return jnp.sum(h2 * lwv[:][None, :], axis=1) + lb


def _vp_body(lbs_ref, x_ref, out0_ref, *refs):
    wrefs = refs[:27]
    dur_ref, pitch_ref, energy_ref = refs[27:]
    dur_ref[0, 0, :] = _vp(x_ref[0], wrefs[0:9], lbs_ref[0])
    ob = out0_ref[0]
    pitch_ref[0, 0, :] = _vp(ob, wrefs[9:18], lbs_ref[1])
    energy_ref[0, 0, :] = _vp(ob, wrefs[18:27], lbs_ref[2])


def _const_spec(shape):
    n = len(shape)
    return pl.BlockSpec(shape, lambda b, _n=n: (0,) * _n)


_WNAMES = ('c1w', 'c1b', 'ln1g', 'ln1b', 'c2w', 'c2b', 'ln2g', 'ln2b')


def kernel(x, target, p_target, e_target, mel_max_length, params, pitch_bins, energy_bins):
    f32 = jnp.float32
    tgt3 = target.astype(f32).reshape(_NB, 1, _LP)
    mel = jnp.asarray(mel_max_length, jnp.int32).reshape(1)
    pt3 = p_target.reshape(_NB, 1, _LM)
    et3 = e_target.reshape(_NB, 1, _LM)
    inf1 = jnp.full((1,), jnp.inf, f32)
    pbins = jnp.concatenate([pitch_bins.astype(f32), inf1])
    ebins = jnp.concatenate([energy_bins.astype(f32), inf1])

    src3, pidx3, eidx3 = pl.pallas_call(
        _idx_body,
        grid=(_NB,),
        in_specs=[
            pl.BlockSpec(memory_space=pltpu.SMEM),
            pl.BlockSpec((1, 1, _LP), lambda b: (b, 0, 0)),
            pl.BlockSpec((1, 1, _LM), lambda b: (b, 0, 0)),
            pl.BlockSpec((1, 1, _LM), lambda b: (b, 0, 0)),
            _const_spec((_NBINS,)),
            _const_spec((_NBINS,)),
        ],
        out_specs=[pl.BlockSpec((1, 1, _LM), lambda b: (b, 0, 0))] * 3,
        out_shape=[jax.ShapeDtypeStruct((_NB, 1, _LM), jnp.int32)] * 3,
    )(mel, tgt3, pt3, et3, pbins, ebins)

    xz = jnp.concatenate([x.reshape(_NB * _LP, _ENC),
                          jnp.zeros((8, _ENC), f32)], axis=0)
    out0f, outf = _sc_gather_fn()(xz, params['pitch_emb'], params['energy_emb'],
                                  src3.reshape(_ROWS), pidx3.reshape(_ROWS),
                                  eidx3.reshape(_ROWS))
    out0 = out0f.reshape(_NB, _LM, _ENC)

    weights = []
    wspecs = []
    for vp in ('dur', 'pitch', 'energy'):
        p = params[vp]
        for nm in _WNAMES:
            weights.append(p[nm])
            wspecs.append(_const_spec(p[nm].shape))
        weights.append(p['lw'][:, 0])
        wspecs.append(_const_spec((_ENC,)))
    lbs = jnp.stack([params['dur']['lb'][0], params['pitch']['lb'][0],
                     params['energy']['lb'][0]])

    dur3, pitch3, energy3 = pl.pallas_call(
        _vp_body,
        grid=(_NB,),
        in_specs=[
            pl.BlockSpec(memory_space=pltpu.SMEM),
            pl.BlockSpec((1, _LP, _ENC), lambda b: (b, 0, 0)),
            pl.BlockSpec((1, _LM, _ENC), lambda b: (b, 0, 0)),
        ] + wspecs,
        out_specs=[
            pl.BlockSpec((1, 1, _LP), lambda b: (b, 0, 0)),
            pl.BlockSpec((1, 1, _LM), lambda b: (b, 0, 0)),
            pl.BlockSpec((1, 1, _LM), lambda b: (b, 0, 0)),
        ],
        out_shape=[
            jax.ShapeDtypeStruct((_NB, 1, _LP), f32),
            jax.ShapeDtypeStruct((_NB, 1, _LM), f32),
            jax.ShapeDtypeStruct((_NB, 1, _LM), f32),
        ],
    )(lbs, x, out0, *weights)

    return (outf.reshape(_NB, _LM, _ENC), dur3.reshape(_NB, _LP),
            pitch3.reshape(_NB, _LM), energy3.reshape(_NB, _LM))


# speedup vs baseline: 4.5518x; 1.1001x over previous
import functools

import jax
import jax.numpy as jnp
from jax import lax
from jax.experimental import pallas as pl
from jax.experimental.pallas import tpu as pltpu
from jax.experimental.pallas import tpu_sc as plsc

_ENC = 256
_NBINS = 256
_NB = 4
_LP = 512
_LM = 1536
_ZROW = _NB * _LP
_XROWS = _NB * _LP + 128
_ROWS = _NB * _LM
_NW = 32
_RPW = _ROWS // _NW
_CH = 96
_NCH = _RPW // _CH


def _idx_body(mel_ref, tgt_ref, pt_ref, et_ref, pbins_ref, ebins_ref,
              src_ref, pidx_ref, eidx_ref):
    b = pl.program_id(0)
    dur = tgt_ref[0, 0, :].reshape(1, _LP)
    tri = (lax.broadcasted_iota(jnp.int32, (_LP, _LP), 0)
           <= lax.broadcasted_iota(jnp.int32, (_LP, _LP), 1)).astype(jnp.float32)
    c = jnp.dot(dur, tri, preferred_element_type=jnp.float32).astype(jnp.int32)
    tot = jnp.minimum(c[0, _LP - 1], mel_ref[0])
    j = lax.broadcasted_iota(jnp.int32, (_LM, 1), 0)
    cnt = jnp.sum((c <= j).astype(jnp.int32), axis=1)
    valid = j[:, 0] < tot
    src_ref[0, 0, :] = jnp.where(valid, b * _LP + cnt, _ZROW)
    pt = pt_ref[0, 0, :]
    pidx_ref[0, 0, :] = jnp.sum((pbins_ref[:][None, :] < pt[:, None]).astype(jnp.int32), axis=1)
    et = et_ref[0, 0, :]
    eidx_ref[0, 0, :] = jnp.sum((ebins_ref[:][None, :] < et[:, None]).astype(jnp.int32), axis=1)


@functools.lru_cache(maxsize=1)
def _sc_gather_fn():
    mesh = plsc.VectorSubcoreMesh(core_axis_name="c", subcore_axis_name="s")

    @functools.partial(
        pl.kernel,
        mesh=mesh,
        out_type=[
            jax.ShapeDtypeStruct((_ROWS, _ENC), jnp.float32),
            jax.ShapeDtypeStruct((_ROWS, _ENC), jnp.float32),
        ],
        scratch_types=[
            pltpu.VMEM((_CH,), jnp.int32),
            pltpu.VMEM((_CH,), jnp.int32),
            pltpu.VMEM((_CH,), jnp.int32),
            pltpu.VMEM((_CH, _ENC), jnp.float32),
            pltpu.VMEM((_CH, _ENC), jnp.float32),
            pltpu.VMEM((_CH, _ENC), jnp.float32),
            pltpu.SemaphoreType.DMA,
        ],
    )
    def _sc_gather(xz, ptab, etab, src, pidx, eidx, out0, outf,
                   src_v, pidx_v, eidx_v, xr, pr, er, sem):
        _sc_gather_body(xz, ptab, etab, src, pidx, eidx, out0, outf,
                        src_v, pidx_v, eidx_v, xr, pr, er, sem)

    return _sc_gather


def _sc_gather_body(xz, ptab, etab, src, pidx, eidx, out0, outf,
                    src_v, pidx_v, eidx_v, xr, pr, er, sem):
    wid = lax.axis_index("s") * 2 + lax.axis_index("c")
    base = wid * _RPW

    def chunk(k, carry):
        off = base + k * _CH
        pltpu.sync_copy(src.at[pl.ds(off, _CH)], src_v)
        pltpu.sync_copy(pidx.at[pl.ds(off, _CH)], pidx_v)
        pltpu.sync_copy(eidx.at[pl.ds(off, _CH)], eidx_v)
        cps = []
        for t in range(_CH // 16):
            s16 = pl.ds(t * 16, 16)
            cps.append(pltpu.async_copy(xz.at[src_v.at[s16]], xr.at[s16], sem))
            cps.append(pltpu.async_copy(ptab.at[pidx_v.at[s16]], pr.at[s16], sem))
            cps.append(pltpu.async_copy(etab.at[eidx_v.at[s16]], er.at[s16], sem))
        for c in cps:
            c.wait()
        pltpu.sync_copy(xr, out0.at[pl.ds(off, _CH)])

        def row(r, carry2):
            for q in range(_ENC // 16):
                sl = pl.ds(q * 16, 16)
                xr[r, sl] = xr[r, sl] + pr[r, sl] + er[r, sl]
            return carry2

        lax.fori_loop(0, _CH, row, 0)
        pltpu.sync_copy(xr, outf.at[pl.ds(off, _CH)])
        return carry

    lax.fori_loop(0, _NCH, chunk, 0)


def _conv_relu_ln(h, w_ref, b_ref, g_ref, bt_ref):
    a0 = jnp.dot(h, w_ref[0], preferred_element_type=jnp.float32)
    a1 = jnp.dot(h, w_ref[1], preferred_element_type=jnp.float32)
    a2 = jnp.dot(h, w_ref[2], preferred_element_type=jnp.float32)
    z = jnp.zeros((1, _ENC), jnp.float32)
    y = (a1 + jnp.concatenate([z, a0[:-1]], axis=0)
         + jnp.concatenate([a2[1:], z], axis=0) + b_ref[:][None, :])
    y = jnp.maximum(y, 0.0)
    m = jnp.mean(y, axis=1, keepdims=True)
    v = jnp.mean(jnp.square(y - m), axis=1, keepdims=True)
    return (y - m) * lax.rsqrt(v + 1e-5) * g_ref[:][None, :] + bt_ref[:][None, :]


def _vp(h, refs, lb):
    c1w, c1b, l1g, l1b, c2w, c2b, l2g, l2b, lwv = refs
    h1 = _conv_relu_ln(h, c1w, c1b, l1g, l1b)
    h2 = _conv_relu_ln(h1, c2w, c2b, l2g, l2b)
    return jnp.sum(h2 * lwv[:][None, :], axis=1) + lb


def _vp_body(lbs_ref, x_ref, out0_ref, *refs):
    wrefs = refs[:27]
    dur_ref, pitch_ref, energy_ref = refs[27:]
    dur_ref[0, 0, :] = _vp(x_ref[0], wrefs[0:9], lbs_ref[0])
    ob = out0_ref[0]
    pitch_ref[0, 0, :] = _vp(ob, wrefs[9:18], lbs_ref[1])
    energy_ref[0, 0, :] = _vp(ob, wrefs[18:27], lbs_ref[2])


def _const_spec(shape):
    n = len(shape)
    return pl.BlockSpec(shape, lambda b, _n=n: (0,) * _n)


_WNAMES = ('c1w', 'c1b', 'ln1g', 'ln1b', 'c2w', 'c2b', 'ln2g', 'ln2b')


def kernel(x, target, p_target, e_target, mel_max_length, params, pitch_bins, energy_bins):
    f32 = jnp.float32
    tgt3 = target.astype(f32).reshape(_NB, 1, _LP)
    mel = jnp.asarray(mel_max_length, jnp.int32).reshape(1)
    pt3 = p_target.reshape(_NB, 1, _LM)
    et3 = e_target.reshape(_NB, 1, _LM)
    inf1 = jnp.full((1,), jnp.inf, f32)
    pbins = jnp.concatenate([pitch_bins.astype(f32), inf1])
    ebins = jnp.concatenate([energy_bins.astype(f32), inf1])

    src3, pidx3, eidx3 = pl.pallas_call(
        _idx_body,
        grid=(_NB,),
        in_specs=[
            pl.BlockSpec(memory_space=pltpu.SMEM),
            pl.BlockSpec((1, 1, _LP), lambda b: (b, 0, 0)),
            pl.BlockSpec((1, 1, _LM), lambda b: (b, 0, 0)),
            pl.BlockSpec((1, 1, _LM), lambda b: (b, 0, 0)),
            _const_spec((_NBINS,)),
            _const_spec((_NBINS,)),
        ],
        out_specs=[pl.BlockSpec((1, 1, _LM), lambda b: (b, 0, 0))] * 3,
        out_shape=[jax.ShapeDtypeStruct((_NB, 1, _LM), jnp.int32)] * 3,
    )(mel, tgt3, pt3, et3, pbins, ebins)

    xz = jnp.concatenate([x.reshape(_NB * _LP, _ENC),
                          jnp.zeros((_XROWS - _NB * _LP, _ENC), f32)], axis=0)
    out0f, outf = _sc_gather_fn()(xz, params['pitch_emb'], params['energy_emb'],
                                  src3.reshape(_ROWS), pidx3.reshape(_ROWS),
                                  eidx3.reshape(_ROWS))
    out0 = out0f.reshape(_NB, _LM, _ENC)

    weights = []
    wspecs = []
    for vp in ('dur', 'pitch', 'energy'):
        p = params[vp]
        for nm in _WNAMES:
            weights.append(p[nm])
            wspecs.append(_const_spec(p[nm].shape))
        weights.append(p['lw'][:, 0])
        wspecs.append(_const_spec((_ENC,)))
    lbs = jnp.stack([params['dur']['lb'][0], params['pitch']['lb'][0],
                     params['energy']['lb'][0]])

    dur3, pitch3, energy3 = pl.pallas_call(
        _vp_body,
        grid=(_NB,),
        in_specs=[
            pl.BlockSpec(memory_space=pltpu.SMEM),
            pl.BlockSpec((1, _LP, _ENC), lambda b: (b, 0, 0)),
            pl.BlockSpec((1, _LM, _ENC), lambda b: (b, 0, 0)),
        ] + wspecs,
        out_specs=[
            pl.BlockSpec((1, 1, _LP), lambda b: (b, 0, 0)),
            pl.BlockSpec((1, 1, _LM), lambda b: (b, 0, 0)),
            pl.BlockSpec((1, 1, _LM), lambda b: (b, 0, 0)),
        ],
        out_shape=[
            jax.ShapeDtypeStruct((_NB, 1, _LP), f32),
            jax.ShapeDtypeStruct((_NB, 1, _LM), f32),
            jax.ShapeDtypeStruct((_NB, 1, _LM), f32),
        ],
    )(lbs, x, out0, *weights)

    return (outf.reshape(_NB, _LM, _ENC), dur3.reshape(_NB, _LP),
            pitch3.reshape(_NB, _LM), energy3.reshape(_NB, _LM))


# speedup vs baseline: 9.5000x; 2.0871x over previous
import functools

import jax
import jax.numpy as jnp
from jax import lax
from jax.experimental import pallas as pl
from jax.experimental.pallas import tpu as pltpu
from jax.experimental.pallas import tpu_sc as plsc

_ENC = 256
_NBINS = 256
_NB = 4
_LP = 512
_LM = 1536
_ROWS = _NB * _LM
_NW = 32
_RPW = _ROWS // _NW


def _idx_body(tgt_ref, src_ref):
    b = pl.program_id(0)
    dur = tgt_ref[0, 0, :].reshape(1, _LP)
    tri = (lax.broadcasted_iota(jnp.int32, (_LP, _LP), 0)
           <= lax.broadcasted_iota(jnp.int32, (_LP, _LP), 1)).astype(jnp.float32)
    c = jnp.dot(dur, tri, preferred_element_type=jnp.float32).astype(jnp.int32)
    j = lax.broadcasted_iota(jnp.int32, (_LM, 1), 0)
    cnt = jnp.sum((c <= j).astype(jnp.int32), axis=1)
    src_ref[0, 0, :] = b * _LP + jnp.minimum(cnt, _LP - 1)


@functools.lru_cache(maxsize=1)
def _sc_gather_fn():
    mesh = plsc.VectorSubcoreMesh(core_axis_name="c", subcore_axis_name="s")

    @functools.partial(
        pl.kernel,
        mesh=mesh,
        out_type=jax.ShapeDtypeStruct((_ROWS, _ENC), jnp.float32),
        scratch_types=[
            pltpu.VMEM((_RPW,), jnp.int32),
            pltpu.VMEM((_RPW, _ENC), jnp.float32),
            pltpu.SemaphoreType.DMA,
        ],
    )
    def _sc_gather(x2, src, out0, src_v, xr, sem):
        wid = lax.axis_index("s") * 2 + lax.axis_index("c")
        base = wid * _RPW
        pltpu.sync_copy(src.at[pl.ds(base, _RPW)], src_v)
        pltpu.async_copy(x2.at[src_v], xr, sem).wait()
        pltpu.sync_copy(xr, out0.at[pl.ds(base, _RPW)])

    return _sc_gather


def _conv_relu_ln(h, w_ref, b_ref, g_ref, bt_ref):
    a0 = jnp.dot(h, w_ref[0], preferred_element_type=jnp.float32)
    a1 = jnp.dot(h, w_ref[1], preferred_element_type=jnp.float32)
    a2 = jnp.dot(h, w_ref[2], preferred_element_type=jnp.float32)
    z = jnp.zeros((1, _ENC), jnp.float32)
    y = (a1 + jnp.concatenate([z, a0[:-1]], axis=0)
         + jnp.concatenate([a2[1:], z], axis=0) + b_ref[:][None, :])
    y = jnp.maximum(y, 0.0)
    m = jnp.mean(y, axis=1, keepdims=True)
    v = jnp.mean(jnp.square(y - m), axis=1, keepdims=True)
    return (y - m) * lax.rsqrt(v + 1e-5) * g_ref[:][None, :] + bt_ref[:][None, :]


def _vp(h, refs, lb):
    c1w, c1b, l1g, l1b, c2w, c2b, l2g, l2b, lwv = refs
    h1 = _conv_relu_ln(h, c1w, c1b, l1g, l1b)
    h2 = _conv_relu_ln(h1, c2w, c2b, l2g, l2b)
    return jnp.sum(h2 * lwv[:][None, :], axis=1) + lb


def _onehot(v, lo_ref, hi_ref):
    return ((lo_ref[:][None, :] < v[:, None])
            & (v[:, None] <= hi_ref[:][None, :])).astype(jnp.float32)


def _vp_body(lbs_ref, mel_ref, x_ref, out0_ref, tgt_ref, pt_ref, et_ref,
             plo_ref, phi_ref, elo_ref, ehi_ref, ptab_ref, etab_ref, *refs):
    wrefs = refs[:27]
    outf_ref, dur_ref, pitch_ref, energy_ref = refs[27:]

    tot = jnp.sum(tgt_ref[0, 0, :]).astype(jnp.int32)
    lim = jnp.minimum(tot, mel_ref[0])
    mask = (lax.broadcasted_iota(jnp.int32, (_LM, 1), 0) < lim).astype(jnp.float32)
    ob = out0_ref[0] * mask

    emb = (jnp.dot(_onehot(pt_ref[0, 0, :], plo_ref, phi_ref), ptab_ref[:],
                   preferred_element_type=jnp.float32)
           + jnp.dot(_onehot(et_ref[0, 0, :], elo_ref, ehi_ref), etab_ref[:],
                     preferred_element_type=jnp.float32))
    outf_ref[0] = ob + emb

    dur_ref[0, 0, :] = _vp(x_ref[0], wrefs[0:9], lbs_ref[0])
    pitch_ref[0, 0, :] = _vp(ob, wrefs[9:18], lbs_ref[1])
    energy_ref[0, 0, :] = _vp(ob, wrefs[18:27], lbs_ref[2])


def _const_spec(shape):
    n = len(shape)
    return pl.BlockSpec(shape, lambda b, _n=n: (0,) * _n)


_WNAMES = ('c1w', 'c1b', 'ln1g', 'ln1b', 'c2w', 'c2b', 'ln2g', 'ln2b')


def kernel(x, target, p_target, e_target, mel_max_length, params, pitch_bins, energy_bins):
    f32 = jnp.float32
    tgt3 = target.astype(f32).reshape(_NB, 1, _LP)
    mel = jnp.asarray(mel_max_length, jnp.int32).reshape(1)
    pt3 = p_target.reshape(_NB, 1, _LM)
    et3 = e_target.reshape(_NB, 1, _LM)
    ninf = jnp.full((1,), -jnp.inf, f32)
    pinf = jnp.full((1,), jnp.inf, f32)
    plo = jnp.concatenate([ninf, pitch_bins.astype(f32)])
    phi = jnp.concatenate([pitch_bins.astype(f32), pinf])
    elo = jnp.concatenate([ninf, energy_bins.astype(f32)])
    ehi = jnp.concatenate([energy_bins.astype(f32), pinf])

    src3 = pl.pallas_call(
        _idx_body,
        grid=(_NB,),
        in_specs=[pl.BlockSpec((1, 1, _LP), lambda b: (b, 0, 0))],
        out_specs=pl.BlockSpec((1, 1, _LM), lambda b: (b, 0, 0)),
        out_shape=jax.ShapeDtypeStruct((_NB, 1, _LM), jnp.int32),
    )(tgt3)

    out0f = _sc_gather_fn()(x.reshape(_NB * _LP, _ENC), src3.reshape(_ROWS))
    out0 = out0f.reshape(_NB, _LM, _ENC)

    weights = []
    wspecs = []
    for vp in ('dur', 'pitch', 'energy'):
        p = params[vp]
        for nm in _WNAMES:
            weights.append(p[nm])
            wspecs.append(_const_spec(p[nm].shape))
        weights.append(p['lw'][:, 0])
        wspecs.append(_const_spec((_ENC,)))
    lbs = jnp.stack([params['dur']['lb'][0], params['pitch']['lb'][0],
                     params['energy']['lb'][0]])

    outf, dur3, pitch3, energy3 = pl.pallas_call(
        _vp_body,
        grid=(_NB,),
        in_specs=[
            pl.BlockSpec(memory_space=pltpu.SMEM),
            pl.BlockSpec(memory_space=pltpu.SMEM),
            pl.BlockSpec((1, _LP, _ENC), lambda b: (b, 0, 0)),
            pl.BlockSpec((1, _LM, _ENC), lambda b: (b, 0, 0)),
            pl.BlockSpec((1, 1, _LP), lambda b: (b, 0, 0)),
            pl.BlockSpec((1, 1, _LM), lambda b: (b, 0, 0)),
            pl.BlockSpec((1, 1, _LM), lambda b: (b, 0, 0)),
            _const_spec((_NBINS,)),
            _const_spec((_NBINS,)),
            _const_spec((_NBINS,)),
            _const_spec((_NBINS,)),
            _const_spec((_NBINS, _ENC)),
            _const_spec((_NBINS, _ENC)),
        ] + wspecs,
        out_specs=[
            pl.BlockSpec((1, _LM, _ENC), lambda b: (b, 0, 0)),
            pl.BlockSpec((1, 1, _LP), lambda b: (b, 0, 0)),
            pl.BlockSpec((1, 1, _LM), lambda b: (b, 0, 0)),
            pl.BlockSpec((1, 1, _LM), lambda b: (b, 0, 0)),
        ],
        out_shape=[
            jax.ShapeDtypeStruct((_NB, _LM, _ENC), f32),
            jax.ShapeDtypeStruct((_NB, 1, _LP), f32),
            jax.ShapeDtypeStruct((_NB, 1, _LM), f32),
            jax.ShapeDtypeStruct((_NB, 1, _LM), f32),
        ],
    )(lbs, mel, x, out0, tgt3, pt3, et3, plo, phi, elo, ehi,
      params['pitch_emb'], params['energy_emb'], *weights)

    return (outf, dur3.reshape(_NB, _LP),
            pitch3.reshape(_NB, _LM), energy3.reshape(_NB, _LM))


# speedup vs baseline: 12.7351x; 1.3405x over previous
import functools

import jax
import jax.numpy as jnp
from jax import lax
from jax.experimental import pallas as pl
from jax.experimental.pallas import tpu as pltpu
from jax.experimental.pallas import tpu_sc as plsc

_ENC = 256
_NBINS = 256
_NB = 4
_LP = 512
_LM = 1536
_ROWS = _NB * _LM
_NW = 32
_RPW = _ROWS // _NW


def _idx_body(mel_ref, tgt_ref, src_ref, lim_ref):
    b = pl.program_id(0)
    dur = tgt_ref[0, 0, :].reshape(1, _LP)
    tri = (lax.broadcasted_iota(jnp.int32, (_LP, _LP), 0)
           <= lax.broadcasted_iota(jnp.int32, (_LP, _LP), 1)).astype(jnp.float32)
    c = jnp.dot(dur, tri, preferred_element_type=jnp.float32).astype(jnp.int32)
    j = lax.broadcasted_iota(jnp.int32, (_LM, 1), 0)
    cnt = jnp.sum((c <= j).astype(jnp.int32), axis=1)
    src_ref[0, 0, :] = b * _LP + jnp.minimum(cnt, _LP - 1)
    lim = jnp.minimum(c[0, _LP - 1], mel_ref[0])
    lim_ref[0, 0, :] = jnp.full((128,), lim, jnp.int32).astype(jnp.float32)


@functools.lru_cache(maxsize=1)
def _sc_gather_fn():
    mesh = plsc.VectorSubcoreMesh(core_axis_name="c", subcore_axis_name="s")

    @functools.partial(
        pl.kernel,
        mesh=mesh,
        out_type=jax.ShapeDtypeStruct((_ROWS, _ENC), jnp.float32),
        scratch_types=[
            pltpu.VMEM((_RPW,), jnp.int32),
            pltpu.VMEM((_RPW, _ENC), jnp.float32),
            pltpu.VMEM((16,), jnp.float32),
            pltpu.SemaphoreType.DMA,
        ],
    )
    def _sc_gather(x2, src, lim, out0, src_v, xr, lim_v, sem):
        wid = lax.axis_index("s") * 2 + lax.axis_index("c")
        b = wid // 8
        k = wid % 8
        pltpu.sync_copy(lim.at[pl.ds(b * 128, 16)], lim_v)
        limv = lim_v[...][0].astype(jnp.int32)
        qb = ((limv + 7) // 8 + 7) // 8 * 8
        jlo = k * qb
        u = jnp.clip(limv - jlo, 0, qb)
        base = b * _LM + jlo
        pltpu.sync_copy(src.at[pl.ds(base, _RPW)], src_v)

        def gb(t, carry):
            s16 = pl.ds(t * 16, 16)
            pltpu.async_copy(x2.at[src_v.at[s16]], xr.at[s16], sem).wait()
            pltpu.sync_copy(xr.at[s16], out0.at[pl.ds(base + t * 16, 16)])
            return carry

        lax.fori_loop(0, (u + 15) // 16, gb, 0)

    return _sc_gather


def _conv_relu_ln(h, w_ref, b_ref, g_ref, bt_ref):
    a0 = jnp.dot(h, w_ref[0], preferred_element_type=jnp.float32)
    a1 = jnp.dot(h, w_ref[1], preferred_element_type=jnp.float32)
    a2 = jnp.dot(h, w_ref[2], preferred_element_type=jnp.float32)
    z = jnp.zeros((1, _ENC), jnp.float32)
    y = (a1 + jnp.concatenate([z, a0[:-1]], axis=0)
         + jnp.concatenate([a2[1:], z], axis=0) + b_ref[:][None, :])
    y = jnp.maximum(y, 0.0)
    m = jnp.mean(y, axis=1, keepdims=True)
    v = jnp.mean(jnp.square(y - m), axis=1, keepdims=True)
    return (y - m) * lax.rsqrt(v + 1e-5) * g_ref[:][None, :] + bt_ref[:][None, :]


def _vp(h, refs, lb):
    c1w, c1b, l1g, l1b, c2w, c2b, l2g, l2b, lwv = refs
    h1 = _conv_relu_ln(h, c1w, c1b, l1g, l1b)
    h2 = _conv_relu_ln(h1, c2w, c2b, l2g, l2b)
    return jnp.sum(h2 * lwv[:][None, :], axis=1) + lb


def _onehot(v, lo_ref, hi_ref):
    return ((lo_ref[:][None, :] < v[:, None])
            & (v[:, None] <= hi_ref[:][None, :])).astype(jnp.float32)


def _vp_body(lbs_ref, mel_ref, x_ref, out0_ref, tgt_ref, pt_ref, et_ref,
             plo_ref, phi_ref, elo_ref, ehi_ref, ptab_ref, etab_ref, *refs):
    wrefs = refs[:27]
    outf_ref, dur_ref, pitch_ref, energy_ref = refs[27:]

    tot = jnp.sum(tgt_ref[0, 0, :]).astype(jnp.int32)
    lim = jnp.minimum(tot, mel_ref[0])
    mvalid = lax.broadcasted_iota(jnp.int32, (_LM, 1), 0) < lim
    ob = jnp.where(mvalid, out0_ref[0], 0.0)

    emb = (jnp.dot(_onehot(pt_ref[0, 0, :], plo_ref, phi_ref), ptab_ref[:],
                   preferred_element_type=jnp.float32)
           + jnp.dot(_onehot(et_ref[0, 0, :], elo_ref, ehi_ref), etab_ref[:],
                     preferred_element_type=jnp.float32))
    outf_ref[0] = ob + emb

    dur_ref[0, 0, :] = _vp(x_ref[0], wrefs[0:9], lbs_ref[0])
    pitch_ref[0, 0, :] = _vp(ob, wrefs[9:18], lbs_ref[1])
    energy_ref[0, 0, :] = _vp(ob, wrefs[18:27], lbs_ref[2])


def _const_spec(shape):
    n = len(shape)
    return pl.BlockSpec(shape, lambda b, _n=n: (0,) * _n)


_WNAMES = ('c1w', 'c1b', 'ln1g', 'ln1b', 'c2w', 'c2b', 'ln2g', 'ln2b')


def kernel(x, target, p_target, e_target, mel_max_length, params, pitch_bins, energy_bins):
    f32 = jnp.float32
    tgt3 = target.astype(f32).reshape(_NB, 1, _LP)
    mel = jnp.asarray(mel_max_length, jnp.int32).reshape(1)
    pt3 = p_target.reshape(_NB, 1, _LM)
    et3 = e_target.reshape(_NB, 1, _LM)
    ninf = jnp.full((1,), -jnp.inf, f32)
    pinf = jnp.full((1,), jnp.inf, f32)
    plo = jnp.concatenate([ninf, pitch_bins.astype(f32)])
    phi = jnp.concatenate([pitch_bins.astype(f32), pinf])
    elo = jnp.concatenate([ninf, energy_bins.astype(f32)])
    ehi = jnp.concatenate([energy_bins.astype(f32), pinf])

    src3, lim3 = pl.pallas_call(
        _idx_body,
        grid=(_NB,),
        in_specs=[
            pl.BlockSpec(memory_space=pltpu.SMEM),
            pl.BlockSpec((1, 1, _LP), lambda b: (b, 0, 0)),
        ],
        out_specs=[
            pl.BlockSpec((1, 1, _LM), lambda b: (b, 0, 0)),
            pl.BlockSpec((1, 1, 128), lambda b: (b, 0, 0)),
        ],
        out_shape=[
            jax.ShapeDtypeStruct((_NB, 1, _LM), jnp.int32),
            jax.ShapeDtypeStruct((_NB, 1, 128), jnp.float32),
        ],
    )(mel, tgt3)

    out0f = _sc_gather_fn()(x.reshape(_NB * _LP, _ENC), src3.reshape(_ROWS),
                            lim3.reshape(_NB * 128))
    out0 = out0f.reshape(_NB, _LM, _ENC)

    weights = []
    wspecs = []
    for vp in ('dur', 'pitch', 'energy'):
        p = params[vp]
        for nm in _WNAMES:
            weights.append(p[nm])
            wspecs.append(_const_spec(p[nm].shape))
        weights.append(p['lw'][:, 0])
        wspecs.append(_const_spec((_ENC,)))
    lbs = jnp.stack([params['dur']['lb'][0], params['pitch']['lb'][0],
                     params['energy']['lb'][0]])

    outf, dur3, pitch3, energy3 = pl.pallas_call(
        _vp_body,
        grid=(_NB,),
        in_specs=[
            pl.BlockSpec(memory_space=pltpu.SMEM),
            pl.BlockSpec(memory_space=pltpu.SMEM),
            pl.BlockSpec((1, _LP, _ENC), lambda b: (b, 0, 0)),
            pl.BlockSpec((1, _LM, _ENC), lambda b: (b, 0, 0)),
            pl.BlockSpec((1, 1, _LP), lambda b: (b, 0, 0)),
            pl.BlockSpec((1, 1, _LM), lambda b: (b, 0, 0)),
            pl.BlockSpec((1, 1, _LM), lambda b: (b, 0, 0)),
            _const_spec((_NBINS,)),
            _const_spec((_NBINS,)),
            _const_spec((_NBINS,)),
            _const_spec((_NBINS,)),
            _const_spec((_NBINS, _ENC)),
            _const_spec((_NBINS, _ENC)),
        ] + wspecs,
        out_specs=[
            pl.BlockSpec((1, _LM, _ENC), lambda b: (b, 0, 0)),
            pl.BlockSpec((1, 1, _LP), lambda b: (b, 0, 0)),
            pl.BlockSpec((1, 1, _LM), lambda b: (b, 0, 0)),
            pl.BlockSpec((1, 1, _LM), lambda b: (b, 0, 0)),
        ],
        out_shape=[
            jax.ShapeDtypeStruct((_NB, _LM, _ENC), f32),
            jax.ShapeDtypeStruct((_NB, 1, _LP), f32),
            jax.ShapeDtypeStruct((_NB, 1, _LM), f32),
            jax.ShapeDtypeStruct((_NB, 1, _LM), f32),
        ],
    )(lbs, mel, x, out0, tgt3, pt3, et3, plo, phi, elo, ehi,
      params['pitch_emb'], params['energy_emb'], *weights)

    return (outf, dur3.reshape(_NB, _LP),
            pitch3.reshape(_NB, _LM), energy3.reshape(_NB, _LM))


# speedup vs baseline: 12.7553x; 1.0016x over previous
import functools

import jax
import jax.numpy as jnp
from jax import lax
from jax.experimental import pallas as pl
from jax.experimental.pallas import tpu as pltpu
from jax.experimental.pallas import tpu_sc as plsc

_ENC = 256
_NBINS = 256
_NB = 4
_LP = 512
_LM = 1536
_ROWS = _NB * _LM
_NW = 32
_RPW = _ROWS // _NW


def _idx_body(mel_ref, tgt_ref, src_ref, lim_ref):
    b = pl.program_id(0)
    dur = tgt_ref[0, 0, :].reshape(1, _LP)
    tri = (lax.broadcasted_iota(jnp.int32, (_LP, _LP), 0)
           <= lax.broadcasted_iota(jnp.int32, (_LP, _LP), 1)).astype(jnp.float32)
    c = jnp.dot(dur, tri, preferred_element_type=jnp.float32).astype(jnp.int32)
    j = lax.broadcasted_iota(jnp.int32, (_LM, 1), 0)
    cnt = jnp.sum((c <= j).astype(jnp.int32), axis=1)
    src_ref[0, 0, :] = b * _LP + jnp.minimum(cnt, _LP - 1)
    lim = jnp.minimum(c[0, _LP - 1], mel_ref[0])
    lim_ref[0, 0, :] = jnp.full((128,), lim, jnp.int32).astype(jnp.float32)


@functools.lru_cache(maxsize=1)
def _sc_gather_fn():
    mesh = plsc.VectorSubcoreMesh(core_axis_name="c", subcore_axis_name="s")

    @functools.partial(
        pl.kernel,
        mesh=mesh,
        out_type=jax.ShapeDtypeStruct((_ROWS, _ENC), jnp.float32),
        scratch_types=[
            pltpu.VMEM((_RPW,), jnp.int32),
            pltpu.VMEM((_RPW, _ENC), jnp.float32),
            pltpu.VMEM((16,), jnp.float32),
            pltpu.SemaphoreType.DMA,
        ],
    )
    def _sc_gather(x2, src, lim, out0, src_v, xr, lim_v, sem):
        wid = lax.axis_index("s") * 2 + lax.axis_index("c")
        b = wid // 8
        k = wid % 8
        pltpu.sync_copy(lim.at[pl.ds(b * 128, 16)], lim_v)
        limv = lim_v[...][0].astype(jnp.int32)
        qb = ((limv + 7) // 8 + 7) // 8 * 8
        jlo = k * qb
        u = jnp.clip(limv - jlo, 0, qb)
        base = b * _LM + jlo
        pltpu.sync_copy(src.at[pl.ds(base, _RPW)], src_v)

        def gb(t, carry):
            s16 = pl.ds(t * 16, 16)
            pltpu.async_copy(x2.at[src_v.at[s16]], xr.at[s16], sem).wait()
            pltpu.sync_copy(xr.at[s16], out0.at[pl.ds(base + t * 16, 16)])
            return carry

        lax.fori_loop(0, (u + 15) // 16, gb, 0)

    return _sc_gather


def _conv_relu_ln(h, w_ref, b_ref, g_ref, bt_ref):
    a0 = jnp.dot(h, w_ref[0], preferred_element_type=jnp.float32)
    a1 = jnp.dot(h, w_ref[1], preferred_element_type=jnp.float32)
    a2 = jnp.dot(h, w_ref[2], preferred_element_type=jnp.float32)
    z = jnp.zeros((1, _ENC), jnp.float32)
    y = (a1 + jnp.concatenate([z, a0[:-1]], axis=0)
         + jnp.concatenate([a2[1:], z], axis=0) + b_ref[:][None, :])
    y = jnp.maximum(y, 0.0)
    m = jnp.mean(y, axis=1, keepdims=True)
    v = jnp.mean(jnp.square(y - m), axis=1, keepdims=True)
    return (y - m) * lax.rsqrt(v + 1e-5) * g_ref[:][None, :] + bt_ref[:][None, :]


def _vp(h, refs, lb):
    c1w, c1b, l1g, l1b, c2w, c2b, l2g, l2b, lwv = refs
    h1 = _conv_relu_ln(h, c1w, c1b, l1g, l1b)
    h2 = _conv_relu_ln(h1, c2w, c2b, l2g, l2b)
    return jnp.sum(h2 * lwv[:][None, :], axis=1) + lb


def _gemask(v, lo_ref):
    return (lo_ref[:][None, :] < v[:, None]).astype(jnp.float32)


def _vp_body(lbs_ref, mel_ref, x_ref, out0_ref, tgt_ref, pt_ref, et_ref,
             plo_ref, elo_ref, ptab_ref, etab_ref, *refs):
    wrefs = refs[:27]
    outf_ref, dur_ref, pitch_ref, energy_ref = refs[27:]

    tot = jnp.sum(tgt_ref[0, 0, :]).astype(jnp.int32)
    lim = jnp.minimum(tot, mel_ref[0])
    mvalid = lax.broadcasted_iota(jnp.int32, (_LM, 1), 0) < lim
    ob = jnp.where(mvalid, out0_ref[0], 0.0)

    emb = (jnp.dot(_gemask(pt_ref[0, 0, :], plo_ref), ptab_ref[:],
                   preferred_element_type=jnp.float32)
           + jnp.dot(_gemask(et_ref[0, 0, :], elo_ref), etab_ref[:],
                     preferred_element_type=jnp.float32))
    outf_ref[0] = ob + emb

    dur_ref[0, 0, :] = _vp(x_ref[0], wrefs[0:9], lbs_ref[0])
    pitch_ref[0, 0, :] = _vp(ob, wrefs[9:18], lbs_ref[1])
    energy_ref[0, 0, :] = _vp(ob, wrefs[18:27], lbs_ref[2])


def _const_spec(shape):
    n = len(shape)
    return pl.BlockSpec(shape, lambda b, _n=n: (0,) * _n)


_WNAMES = ('c1w', 'c1b', 'ln1g', 'ln1b', 'c2w', 'c2b', 'ln2g', 'ln2b')


def kernel(x, target, p_target, e_target, mel_max_length, params, pitch_bins, energy_bins):
    f32 = jnp.float32
    tgt3 = target.astype(f32).reshape(_NB, 1, _LP)
    mel = jnp.asarray(mel_max_length, jnp.int32).reshape(1)
    pt3 = p_target.reshape(_NB, 1, _LM)
    et3 = e_target.reshape(_NB, 1, _LM)
    ninf = jnp.full((1,), -jnp.inf, f32)
    plo = jnp.concatenate([ninf, pitch_bins.astype(f32)])
    elo = jnp.concatenate([ninf, energy_bins.astype(f32)])
    zrow = jnp.zeros((1, _ENC), f32)
    pdiff = params['pitch_emb'] - jnp.concatenate([zrow, params['pitch_emb'][:-1]], 0)
    ediff = params['energy_emb'] - jnp.concatenate([zrow, params['energy_emb'][:-1]], 0)

    src3, lim3 = pl.pallas_call(
        _idx_body,
        grid=(_NB,),
        in_specs=[
            pl.BlockSpec(memory_space=pltpu.SMEM),
            pl.BlockSpec((1, 1, _LP), lambda b: (b, 0, 0)),
        ],
        out_specs=[
            pl.BlockSpec((1, 1, _LM), lambda b: (b, 0, 0)),
            pl.BlockSpec((1, 1, 128), lambda b: (b, 0, 0)),
        ],
        out_shape=[
            jax.ShapeDtypeStruct((_NB, 1, _LM), jnp.int32),
            jax.ShapeDtypeStruct((_NB, 1, 128), jnp.float32),
        ],
    )(mel, tgt3)

    out0f = _sc_gather_fn()(x.reshape(_NB * _LP, _ENC), src3.reshape(_ROWS),
                            lim3.reshape(_NB * 128))
    out0 = out0f.reshape(_NB, _LM, _ENC)

    weights = []
    wspecs = []
    for vp in ('dur', 'pitch', 'energy'):
        p = params[vp]
        for nm in _WNAMES:
            weights.append(p[nm])
            wspecs.append(_const_spec(p[nm].shape))
        weights.append(p['lw'][:, 0])
        wspecs.append(_const_spec((_ENC,)))
    lbs = jnp.stack([params['dur']['lb'][0], params['pitch']['lb'][0],
                     params['energy']['lb'][0]])

    outf, dur3, pitch3, energy3 = pl.pallas_call(
        _vp_body,
        grid=(_NB,),
        in_specs=[
            pl.BlockSpec(memory_space=pltpu.SMEM),
            pl.BlockSpec(memory_space=pltpu.SMEM),
            pl.BlockSpec((1, _LP, _ENC), lambda b: (b, 0, 0)),
            pl.BlockSpec((1, _LM, _ENC), lambda b: (b, 0, 0)),
            pl.BlockSpec((1, 1, _LP), lambda b: (b, 0, 0)),
            pl.BlockSpec((1, 1, _LM), lambda b: (b, 0, 0)),
            pl.BlockSpec((1, 1, _LM), lambda b: (b, 0, 0)),
            _const_spec((_NBINS,)),
            _const_spec((_NBINS,)),
            _const_spec((_NBINS, _ENC)),
            _const_spec((_NBINS, _ENC)),
        ] + wspecs,
        out_specs=[
            pl.BlockSpec((1, _LM, _ENC), lambda b: (b, 0, 0)),
            pl.BlockSpec((1, 1, _LP), lambda b: (b, 0, 0)),
            pl.BlockSpec((1, 1, _LM), lambda b: (b, 0, 0)),
            pl.BlockSpec((1, 1, _LM), lambda b: (b, 0, 0)),
        ],
        out_shape=[
            jax.ShapeDtypeStruct((_NB, _LM, _ENC), f32),
            jax.ShapeDtypeStruct((_NB, 1, _LP), f32),
            jax.ShapeDtypeStruct((_NB, 1, _LM), f32),
            jax.ShapeDtypeStruct((_NB, 1, _LM), f32),
        ],
    )(lbs, mel, x, out0, tgt3, pt3, et3, plo, elo, pdiff, ediff, *weights)

    return (outf, dur3.reshape(_NB, _LP),
            pitch3.reshape(_NB, _LM), energy3.reshape(_NB, _LM))


# speedup vs baseline: 13.8221x; 1.0836x over previous
import functools

import jax
import jax.numpy as jnp
from jax import lax
from jax.experimental import pallas as pl
from jax.experimental.pallas import tpu as pltpu
from jax.experimental.pallas import tpu_sc as plsc

_ENC = 256
_NBINS = 256
_NB = 4
_LP = 512
_LM = 1536
_ROWS = _NB * _LM
_NW = 32
_RPW = _ROWS // _NW


def _idx_body(mel_ref, tgt_ref, src_ref, lim_ref):
    b = pl.program_id(0)
    dur = tgt_ref[0, 0, :].reshape(1, _LP)
    tri = (lax.broadcasted_iota(jnp.int32, (_LP, _LP), 0)
           <= lax.broadcasted_iota(jnp.int32, (_LP, _LP), 1)).astype(jnp.float32)
    c = jnp.dot(dur, tri, preferred_element_type=jnp.float32).astype(jnp.int32)
    j = lax.broadcasted_iota(jnp.int32, (_LM, 1), 0)
    cnt = jnp.sum((c <= j).astype(jnp.int32), axis=1)
    src_ref[0, 0, :] = b * _LP + jnp.minimum(cnt, _LP - 1)
    lim = jnp.minimum(c[0, _LP - 1], mel_ref[0])
    lim_ref[0, 0, :] = jnp.full((128,), lim, jnp.int32).astype(jnp.float32)


@functools.lru_cache(maxsize=1)
def _sc_gather_fn():
    mesh = plsc.VectorSubcoreMesh(core_axis_name="c", subcore_axis_name="s")

    @functools.partial(
        pl.kernel,
        mesh=mesh,
        out_type=jax.ShapeDtypeStruct((_ROWS, _ENC), jnp.float32),
        scratch_types=[
            pltpu.VMEM((_RPW,), jnp.int32),
            pltpu.VMEM((_RPW, _ENC), jnp.float32),
            pltpu.VMEM((16,), jnp.float32),
            pltpu.SemaphoreType.DMA,
        ],
    )
    def _sc_gather(x2, src, lim, out0, src_v, xr, lim_v, sem):
        wid = lax.axis_index("s") * 2 + lax.axis_index("c")
        b = wid // 8
        k = wid % 8
        pltpu.sync_copy(lim.at[pl.ds(b * 128, 16)], lim_v)
        limv = lim_v[...][0].astype(jnp.int32)
        qb = ((limv + 7) // 8 + 7) // 8 * 8
        jlo = k * qb
        u = jnp.clip(limv - jlo, 0, qb)
        base = b * _LM + jlo
        pltpu.sync_copy(src.at[pl.ds(base, _RPW)], src_v)

        def gb(t, carry):
            s16 = pl.ds(t * 16, 16)
            pltpu.async_copy(x2.at[src_v.at[s16]], xr.at[s16], sem).wait()
            pltpu.sync_copy(xr.at[s16], out0.at[pl.ds(base + t * 16, 16)])
            return carry

        lax.fori_loop(0, (u + 15) // 16, gb, 0)

    return _sc_gather


def _conv_relu_ln(h, w_ref, b_ref, g_ref, bt_ref):
    a0 = jnp.dot(h, w_ref[0], preferred_element_type=jnp.float32)
    a1 = jnp.dot(h, w_ref[1], preferred_element_type=jnp.float32)
    a2 = jnp.dot(h, w_ref[2], preferred_element_type=jnp.float32)
    z = jnp.zeros((1, _ENC), jnp.float32)
    y = (a1 + jnp.concatenate([z, a0[:-1]], axis=0)
         + jnp.concatenate([a2[1:], z], axis=0) + b_ref[:][None, :])
    y = jnp.maximum(y, 0.0)
    m = jnp.mean(y, axis=1, keepdims=True)
    v = jnp.mean(jnp.square(y - m), axis=1, keepdims=True)
    return (y - m) * lax.rsqrt(v + 1e-5) * g_ref[:][None, :] + bt_ref[:][None, :]


def _vp(h, refs, lb):
    c1w, c1b, l1g, l1b, c2w, c2b, l2g, l2b, lwp = refs
    h1 = _conv_relu_ln(h, c1w, c1b, l1g, l1b)
    h2 = _conv_relu_ln(h1, c2w, c2b, l2g, l2b)
    return jnp.dot(h2, lwp[:], preferred_element_type=jnp.float32)[:, 0] + lb


def _gemask(v, lo_ref):
    return (lo_ref[:][None, :] < v[:, None]).astype(jnp.float32)


def _vp_body(lbs_ref, mel_ref, x_ref, out0_ref, tgt_ref, pt_ref, et_ref,
             plo_ref, elo_ref, ptab_ref, etab_ref, *refs):
    wrefs = refs[:27]
    outf_ref, dur_ref, pitch_ref, energy_ref = refs[27:]

    tot = jnp.sum(tgt_ref[0, 0, :]).astype(jnp.int32)
    lim = jnp.minimum(tot, mel_ref[0])
    mvalid = lax.broadcasted_iota(jnp.int32, (_LM, 1), 0) < lim
    ob = jnp.where(mvalid, out0_ref[0], 0.0)

    emb = (jnp.dot(_gemask(pt_ref[0, 0, :], plo_ref), ptab_ref[:],
                   preferred_element_type=jnp.float32)
           + jnp.dot(_gemask(et_ref[0, 0, :], elo_ref), etab_ref[:],
                     preferred_element_type=jnp.float32))
    outf_ref[0] = ob + emb

    dur_ref[0, 0, :] = _vp(x_ref[0], wrefs[0:9], lbs_ref[0])
    pitch_ref[0, 0, :] = _vp(ob, wrefs[9:18], lbs_ref[1])
    energy_ref[0, 0, :] = _vp(ob, wrefs[18:27], lbs_ref[2])


def _const_spec(shape):
    n = len(shape)
    return pl.BlockSpec(shape, lambda b, _n=n: (0,) * _n)


_WNAMES = ('c1w', 'c1b', 'ln1g', 'ln1b', 'c2w', 'c2b', 'ln2g', 'ln2b')


def kernel(x, target, p_target, e_target, mel_max_length, params, pitch_bins, energy_bins):
    f32 = jnp.float32
    tgt3 = target.astype(f32).reshape(_NB, 1, _LP)
    mel = jnp.asarray(mel_max_length, jnp.int32).reshape(1)
    pt3 = p_target.reshape(_NB, 1, _LM)
    et3 = e_target.reshape(_NB, 1, _LM)
    ninf = jnp.full((1,), -jnp.inf, f32)
    plo = jnp.concatenate([ninf, pitch_bins.astype(f32)])
    elo = jnp.concatenate([ninf, energy_bins.astype(f32)])
    zrow = jnp.zeros((1, _ENC), f32)
    pdiff = params['pitch_emb'] - jnp.concatenate([zrow, params['pitch_emb'][:-1]], 0)
    ediff = params['energy_emb'] - jnp.concatenate([zrow, params['energy_emb'][:-1]], 0)

    src3, lim3 = pl.pallas_call(
        _idx_body,
        grid=(_NB,),
        in_specs=[
            pl.BlockSpec(memory_space=pltpu.SMEM),
            pl.BlockSpec((1, 1, _LP), lambda b: (b, 0, 0)),
        ],
        out_specs=[
            pl.BlockSpec((1, 1, _LM), lambda b: (b, 0, 0)),
            pl.BlockSpec((1, 1, 128), lambda b: (b, 0, 0)),
        ],
        out_shape=[
            jax.ShapeDtypeStruct((_NB, 1, _LM), jnp.int32),
            jax.ShapeDtypeStruct((_NB, 1, 128), jnp.float32),
        ],
    )(mel, tgt3)

    out0f = _sc_gather_fn()(x.reshape(_NB * _LP, _ENC), src3.reshape(_ROWS),
                            lim3.reshape(_NB * 128))
    out0 = out0f.reshape(_NB, _LM, _ENC)

    weights = []
    wspecs = []
    for vp in ('dur', 'pitch', 'energy'):
        p = params[vp]
        for nm in _WNAMES:
            weights.append(p[nm])
            wspecs.append(_const_spec(p[nm].shape))
        weights.append(jnp.pad(p['lw'], ((0, 0), (0, 127))))
        wspecs.append(_const_spec((_ENC, 128)))
    lbs = jnp.stack([params['dur']['lb'][0], params['pitch']['lb'][0],
                     params['energy']['lb'][0]])

    outf, dur3, pitch3, energy3 = pl.pallas_call(
        _vp_body,
        grid=(_NB,),
        in_specs=[
            pl.BlockSpec(memory_space=pltpu.SMEM),
            pl.BlockSpec(memory_space=pltpu.SMEM),
            pl.BlockSpec((1, _LP, _ENC), lambda b: (b, 0, 0)),
            pl.BlockSpec((1, _LM, _ENC), lambda b: (b, 0, 0)),
            pl.BlockSpec((1, 1, _LP), lambda b: (b, 0, 0)),
            pl.BlockSpec((1, 1, _LM), lambda b: (b, 0, 0)),
            pl.BlockSpec((1, 1, _LM), lambda b: (b, 0, 0)),
            _const_spec((_NBINS,)),
            _const_spec((_NBINS,)),
            _const_spec((_NBINS, _ENC)),
            _const_spec((_NBINS, _ENC)),
        ] + wspecs,
        out_specs=[
            pl.BlockSpec((1, _LM, _ENC), lambda b: (b, 0, 0)),
            pl.BlockSpec((1, 1, _LP), lambda b: (b, 0, 0)),
            pl.BlockSpec((1, 1, _LM), lambda b: (b, 0, 0)),
            pl.BlockSpec((1, 1, _LM), lambda b: (b, 0, 0)),
        ],
        out_shape=[
            jax.ShapeDtypeStruct((_NB, _LM, _ENC), f32),
            jax.ShapeDtypeStruct((_NB, 1, _LP), f32),
            jax.ShapeDtypeStruct((_NB, 1, _LM), f32),
            jax.ShapeDtypeStruct((_NB, 1, _LM), f32),
        ],
    )(lbs, mel, x, out0, tgt3, pt3, et3, plo, elo, pdiff, ediff, *weights)

    return (outf, dur3.reshape(_NB, _LP),
            pitch3.reshape(_NB, _LM), energy3.reshape(_NB, _LM))


# speedup vs baseline: 13.8509x; 1.0021x over previous
import functools

import jax
import jax.numpy as jnp
from jax import lax
from jax.experimental import pallas as pl
from jax.experimental.pallas import tpu as pltpu
from jax.experimental.pallas import tpu_sc as plsc

_ENC = 256
_NBINS = 256
_NB = 4
_LP = 512
_LM = 1536
_ROWS = _NB * _LM
_NW = 32
_RPW = _ROWS // _NW


def _idx_body(mel_ref, tgt_ref, src_ref, lim_ref):
    b = pl.program_id(0)
    dur = tgt_ref[0, 0, :].reshape(1, _LP)
    tri = (lax.broadcasted_iota(jnp.int32, (_LP, _LP), 0)
           <= lax.broadcasted_iota(jnp.int32, (_LP, _LP), 1)).astype(jnp.float32)
    c = jnp.dot(dur, tri, preferred_element_type=jnp.float32).astype(jnp.int32)
    j = lax.broadcasted_iota(jnp.int32, (_LM, 1), 0)
    cnt = jnp.sum((c <= j).astype(jnp.int32), axis=1)
    src_ref[0, 0, :] = b * _LP + jnp.minimum(cnt, _LP - 1)
    lim = jnp.minimum(c[0, _LP - 1], mel_ref[0])
    lim_ref[0, 0, :] = jnp.full((128,), lim, jnp.int32).astype(jnp.float32)


@functools.lru_cache(maxsize=1)
def _sc_gather_fn():
    mesh = plsc.VectorSubcoreMesh(core_axis_name="c", subcore_axis_name="s")

    @functools.partial(
        pl.kernel,
        mesh=mesh,
        out_type=jax.ShapeDtypeStruct((_ROWS, _ENC), jnp.float32),
        scratch_types=[
            pltpu.VMEM((_RPW,), jnp.int32),
            pltpu.VMEM((_RPW, _ENC), jnp.float32),
            pltpu.VMEM((16,), jnp.float32),
            pltpu.SemaphoreType.DMA,
        ],
    )
    def _sc_gather(x2, src, lim, out0, src_v, xr, lim_v, sem):
        wid = lax.axis_index("s") * 2 + lax.axis_index("c")
        b = wid // 8
        k = wid % 8
        pltpu.sync_copy(lim.at[pl.ds(b * 128, 16)], lim_v)
        limv = lim_v[...][0].astype(jnp.int32)
        qb = ((limv + 7) // 8 + 7) // 8 * 8
        jlo = k * qb
        u = jnp.clip(limv - jlo, 0, qb)
        base = b * _LM + jlo
        pltpu.sync_copy(src.at[pl.ds(base, _RPW)], src_v)

        def gb(t, carry):
            s16 = pl.ds(t * 16, 16)
            pltpu.async_copy(x2.at[src_v.at[s16]], xr.at[s16], sem).wait()
            pltpu.sync_copy(xr.at[s16], out0.at[pl.ds(base + t * 16, 16)])
            return carry

        lax.fori_loop(0, (u + 15) // 16, gb, 0)

    return _sc_gather


def _conv_relu_ln(h, w_ref, b_ref, g_ref, bt_ref):
    a0 = jnp.dot(h, w_ref[0], preferred_element_type=jnp.float32)
    a1 = jnp.dot(h, w_ref[1], preferred_element_type=jnp.float32)
    a2 = jnp.dot(h, w_ref[2], preferred_element_type=jnp.float32)
    z = jnp.zeros((1, _ENC), jnp.float32)
    y = (a1 + jnp.concatenate([z, a0[:-1]], axis=0)
         + jnp.concatenate([a2[1:], z], axis=0) + b_ref[:][None, :])
    y = jnp.maximum(y, 0.0)
    m = jnp.mean(y, axis=1, keepdims=True)
    v = jnp.mean(jnp.square(y - m), axis=1, keepdims=True)
    return (y - m) * lax.rsqrt(v + 1e-5) * g_ref[:][None, :] + bt_ref[:][None, :]


def _vp(h, refs, lb):
    c1w, c1b, l1g, l1b, c2w, c2b, l2g, l2b, lwp = refs
    h1 = _conv_relu_ln(h, c1w, c1b, l1g, l1b)
    h2 = _conv_relu_ln(h1, c2w, c2b, l2g, l2b)
    return jnp.dot(h2, lwp[:], preferred_element_type=jnp.float32)[:, 0] + lb


def _gemask(v, lo_ref):
    return (lo_ref[:][None, :] < v[:, None]).astype(jnp.float32)


def _vp_body(lbs_ref, mel_ref, x_ref, out0_ref, tgt_ref, pt_ref, et_ref,
             plo_ref, elo_ref, ptab_ref, etab_ref, *refs):
    wrefs = refs[:27]
    outf_ref, dur_ref, pitch_ref, energy_ref = refs[27:]

    tot = jnp.sum(tgt_ref[0, 0, :]).astype(jnp.int32)
    lim = jnp.minimum(tot, mel_ref[0])
    mvalid = lax.broadcasted_iota(jnp.int32, (_LM, 1), 0) < lim
    ob = jnp.where(mvalid, out0_ref[0], 0.0)

    zr = jnp.zeros((1, _ENC), jnp.float32)
    pd = ptab_ref[:] - jnp.concatenate([zr, ptab_ref[:][:-1]], axis=0)
    ed = etab_ref[:] - jnp.concatenate([zr, etab_ref[:][:-1]], axis=0)
    emb = (jnp.dot(_gemask(pt_ref[0, 0, :], plo_ref), pd,
                   preferred_element_type=jnp.float32)
           + jnp.dot(_gemask(et_ref[0, 0, :], elo_ref), ed,
                     preferred_element_type=jnp.float32))
    outf_ref[0] = ob + emb

    dur_ref[0, 0, :] = _vp(x_ref[0], wrefs[0:9], lbs_ref[0])
    pitch_ref[0, 0, :] = _vp(ob, wrefs[9:18], lbs_ref[1])
    energy_ref[0, 0, :] = _vp(ob, wrefs[18:27], lbs_ref[2])


def _const_spec(shape):
    n = len(shape)
    return pl.BlockSpec(shape, lambda b, _n=n: (0,) * _n)


_WNAMES = ('c1w', 'c1b', 'ln1g', 'ln1b', 'c2w', 'c2b', 'ln2g', 'ln2b')


def kernel(x, target, p_target, e_target, mel_max_length, params, pitch_bins, energy_bins):
    f32 = jnp.float32
    tgt3 = target.astype(f32).reshape(_NB, 1, _LP)
    mel = jnp.asarray(mel_max_length, jnp.int32).reshape(1)
    pt3 = p_target.reshape(_NB, 1, _LM)
    et3 = e_target.reshape(_NB, 1, _LM)
    ninf = jnp.full((1,), -jnp.inf, f32)
    plo = jnp.concatenate([ninf, pitch_bins.astype(f32)])
    elo = jnp.concatenate([ninf, energy_bins.astype(f32)])

    src3, lim3 = pl.pallas_call(
        _idx_body,
        grid=(_NB,),
        in_specs=[
            pl.BlockSpec(memory_space=pltpu.SMEM),
            pl.BlockSpec((1, 1, _LP), lambda b: (b, 0, 0)),
        ],
        out_specs=[
            pl.BlockSpec((1, 1, _LM), lambda b: (b, 0, 0)),
            pl.BlockSpec((1, 1, 128), lambda b: (b, 0, 0)),
        ],
        out_shape=[
            jax.ShapeDtypeStruct((_NB, 1, _LM), jnp.int32),
            jax.ShapeDtypeStruct((_NB, 1, 128), jnp.float32),
        ],
    )(mel, tgt3)

    out0f = _sc_gather_fn()(x.reshape(_NB * _LP, _ENC), src3.reshape(_ROWS),
                            lim3.reshape(_NB * 128))
    out0 = out0f.reshape(_NB, _LM, _ENC)

    weights = []
    wspecs = []
    for vp in ('dur', 'pitch', 'energy'):
        p = params[vp]
        for nm in _WNAMES:
            weights.append(p[nm])
            wspecs.append(_const_spec(p[nm].shape))
        weights.append(jnp.pad(p['lw'], ((0, 0), (0, 127))))
        wspecs.append(_const_spec((_ENC, 128)))
    lbs = jnp.stack([params['dur']['lb'][0], params['pitch']['lb'][0],
                     params['energy']['lb'][0]])

    outf, dur3, pitch3, energy3 = pl.pallas_call(
        _vp_body,
        grid=(_NB,),
        in_specs=[
            pl.BlockSpec(memory_space=pltpu.SMEM),
            pl.BlockSpec(memory_space=pltpu.SMEM),
            pl.BlockSpec((1, _LP, _ENC), lambda b: (b, 0, 0)),
            pl.BlockSpec((1, _LM, _ENC), lambda b: (b, 0, 0)),
            pl.BlockSpec((1, 1, _LP), lambda b: (b, 0, 0)),
            pl.BlockSpec((1, 1, _LM), lambda b: (b, 0, 0)),
            pl.BlockSpec((1, 1, _LM), lambda b: (b, 0, 0)),
            _const_spec((_NBINS,)),
            _const_spec((_NBINS,)),
            _const_spec((_NBINS, _ENC)),
            _const_spec((_NBINS, _ENC)),
        ] + wspecs,
        out_specs=[
            pl.BlockSpec((1, _LM, _ENC), lambda b: (b, 0, 0)),
            pl.BlockSpec((1, 1, _LP), lambda b: (b, 0, 0)),
            pl.BlockSpec((1, 1, _LM), lambda b: (b, 0, 0)),
            pl.BlockSpec((1, 1, _LM), lambda b: (b, 0, 0)),
        ],
        out_shape=[
            jax.ShapeDtypeStruct((_NB, _LM, _ENC), f32),
            jax.ShapeDtypeStruct((_NB, 1, _LP), f32),
            jax.ShapeDtypeStruct((_NB, 1, _LM), f32),
            jax.ShapeDtypeStruct((_NB, 1, _LM), f32),
        ],
    )(lbs, mel, x, out0, tgt3, pt3, et3, plo, elo,
      params['pitch_emb'], params['energy_emb'], *weights)

    return (outf, dur3.reshape(_NB, _LP),
            pitch3.reshape(_NB, _LM), energy3.reshape(_NB, _LM))
